# baseline (speedup 1.0000x reference)
import jax
import jax.numpy as jnp
from jax.experimental import pallas as pl

N = 512
D = 256


def _body(x_ref, pe_ref, o_ref):
    o_ref[...] = x_ref[...] + pe_ref[...][None]


def _half(x, pe_weight, blk_off, rows):
    bi = 16
    return pl.pallas_call(
        _body,
        grid=(rows // bi,),
        in_specs=[
            pl.BlockSpec((bi, N, D), lambda i, o=blk_off: (i + o, 0, 0)),
            pl.BlockSpec((N, D), lambda i: (0, 0)),
        ],
        out_specs=pl.BlockSpec((bi, N, D), lambda i: (i, 0, 0)),
        out_shape=jax.ShapeDtypeStruct((rows, N, D), jnp.float32),
    )(x, pe_weight)


@jax.jit
def kernel(x, pe_weight):
    top = _half(x, pe_weight, 0, 256)
    bot = _half(x, pe_weight, 16, 256)
    return jnp.concatenate([top, bot], axis=0)
